# baseline (speedup 1.0000x reference)
import functools

import jax
import jax.numpy as jnp
from jax import lax
from jax.experimental import pallas as pl
from jax.experimental.pallas import tpu as pltpu
from jax.experimental.pallas import tpu_sc as plsc

ROWS = 16384
COLS = 200
NC = 2
NS = 16
NW = NC * NS
LANES = 16
CBLK = 128
COLS_PER_TILE = ROWS // NW
N_BLKS = COLS_PER_TILE // CBLK


def _lut_body(table_hbm, idx_hbm, out_hbm, tab_v, idx_v, out_v,
              in_sem0, in_sem1, out_sem0, out_sem1):
    in_sems = (in_sem0, in_sem1)
    out_sems = (out_sem0, out_sem1)
    wid = lax.axis_index("s") * NC + lax.axis_index("c")
    base = wid * COLS_PER_TILE
    pltpu.sync_copy(table_hbm, tab_v)

    def in_copy(b):
        col0 = base + b * CBLK
        return pltpu.make_async_copy(
            idx_hbm.at[:, pl.ds(col0, CBLK)], idx_v.at[b % 2], in_sems[b % 2])

    def out_copy(b):
        col0 = base + b * CBLK
        return pltpu.make_async_copy(
            out_v.at[b % 2], out_hbm.at[:, pl.ds(col0, CBLK)], out_sems[b % 2])

    in_copy(0).start()
    for b in range(N_BLKS):
        if b + 1 < N_BLKS:
            in_copy(b + 1).start()
        in_copy(b).wait()
        if b >= 2:
            out_copy(b - 2).wait()

        @plsc.parallel_loop(0, COLS, step=1, unroll=4)
        def row_body(r, _b=b % 2):
            for c in range(0, CBLK, LANES):
                iv = (idx_v[_b, r, pl.ds(c, LANES)] + 128) & 255
                out_v[_b, r, pl.ds(c, LANES)] = plsc.load_gather(tab_v, [iv])

        out_copy(b).start()
    out_copy(N_BLKS - 2).wait()
    out_copy(N_BLKS - 1).wait()


@jax.jit
def _lut(table, idx_t):
    mesh = plsc.VectorSubcoreMesh(core_axis_name="c", subcore_axis_name="s")
    f = functools.partial(
        pl.kernel,
        out_type=jax.ShapeDtypeStruct((COLS, ROWS), jnp.float32),
        mesh=mesh,
        compiler_params=pltpu.CompilerParams(
            needs_layout_passes=False,
            skip_device_barrier=True,
            disable_bounds_checks=True,
            disable_semaphore_checks=True,
        ),
        scratch_types=[
            pltpu.VMEM((256,), jnp.float32),
            pltpu.VMEM((2, COLS, CBLK), jnp.int32),
            pltpu.VMEM((2, COLS, CBLK), jnp.float32),
            pltpu.SemaphoreType.DMA,
            pltpu.SemaphoreType.DMA,
            pltpu.SemaphoreType.DMA,
            pltpu.SemaphoreType.DMA,
        ],
    )(_lut_body)
    return f(table, idx_t)


def kernel(table, index):
    out = _lut(table, index.T).T
    scale = jnp.array([2.0 / 256.0], dtype=jnp.float32)
    return (out, scale)


# speedup vs baseline: 1.0123x; 1.0123x over previous
import functools

import jax
import jax.numpy as jnp
from jax import lax
from jax.experimental import pallas as pl
from jax.experimental.pallas import tpu as pltpu
from jax.experimental.pallas import tpu_sc as plsc

ROWS = 16384
COLS = 200
NC = 2
NS = 16
NW = NC * NS
LANES = 16
CBLK = 128
COLS_PER_TILE = ROWS // NW
N_BLKS = COLS_PER_TILE // CBLK


def _lut_body(table_hbm, idx_hbm, out_hbm, tab_v, idx_v, out_v,
              in_sem0, in_sem1, out_sem0, out_sem1):
    in_sems = (in_sem0, in_sem1)
    out_sems = (out_sem0, out_sem1)
    wid = lax.axis_index("s") * NC + lax.axis_index("c")
    base = wid * COLS_PER_TILE
    pltpu.sync_copy(table_hbm, tab_v)

    def in_copy(b):
        col0 = base + b * CBLK
        return pltpu.make_async_copy(
            idx_hbm.at[:, pl.ds(col0, CBLK)], idx_v.at[b % 2], in_sems[b % 2])

    def out_copy(b):
        col0 = base + b * CBLK
        return pltpu.make_async_copy(
            out_v.at[b % 2], out_hbm.at[:, pl.ds(col0, CBLK)], out_sems[b % 2])

    in_copy(0).start()
    for b in range(N_BLKS):
        if b + 1 < N_BLKS:
            in_copy(b + 1).start()
        in_copy(b).wait()
        if b >= 2:
            out_copy(b - 2).wait()

        @plsc.parallel_loop(0, COLS, step=1, unroll=2)
        def row_body(r, _b=b % 2):
            for c in range(0, CBLK, LANES):
                iv = (idx_v[_b, r, pl.ds(c, LANES)] + 128) & 255
                out_v[_b, r, pl.ds(c, LANES)] = plsc.load_gather(tab_v, [iv])

        out_copy(b).start()
    out_copy(N_BLKS - 2).wait()
    out_copy(N_BLKS - 1).wait()


@jax.jit
def _lut(table, idx_t):
    mesh = plsc.VectorSubcoreMesh(core_axis_name="c", subcore_axis_name="s")
    f = functools.partial(
        pl.kernel,
        out_type=jax.ShapeDtypeStruct((COLS, ROWS), jnp.float32),
        mesh=mesh,
        compiler_params=pltpu.CompilerParams(
            needs_layout_passes=False,
            skip_device_barrier=True,
            disable_bounds_checks=True,
            disable_semaphore_checks=True,
        ),
        scratch_types=[
            pltpu.VMEM((256,), jnp.float32),
            pltpu.VMEM((2, COLS, CBLK), jnp.int32),
            pltpu.VMEM((2, COLS, CBLK), jnp.float32),
            pltpu.SemaphoreType.DMA,
            pltpu.SemaphoreType.DMA,
            pltpu.SemaphoreType.DMA,
            pltpu.SemaphoreType.DMA,
        ],
    )(_lut_body)
    return f(table, idx_t)


def kernel(table, index):
    out = _lut(table, index.T).T
    scale = jnp.array([2.0 / 256.0], dtype=jnp.float32)
    return (out, scale)
